# baseline (speedup 1.0000x reference)
import jax
import jax.numpy as jnp
from jax.experimental import pallas as pl
from jax.experimental.pallas import tpu as pltpu

_G = 4
_ALPHA = 0.5
_K = 4
_NEG = -3.0e38
_R = 8


def _tc_body(lprobs_ref, bias_ref, outs_ref, outi_ref, outb_ref, pen_ref):
    C = lprobs_ref.shape[-1]
    f = (
        jax.lax.broadcasted_iota(jnp.int32, (_R, C), 0) * C
        + jax.lax.broadcasted_iota(jnp.int32, (_R, C), 1)
    )
    col = jax.lax.broadcasted_iota(jnp.int32, (1, C), 1)
    riota = jax.lax.broadcasted_iota(jnp.int32, (_R, 1), 0)
    pen_ref[...] = jnp.zeros((_R, C), jnp.float32)
    BIG = jnp.int32(2**30)
    NEGF = jnp.float32(_NEG)
    oiota = jax.lax.broadcasted_iota(jnp.int32, (1, 1, _G * _K), 2)
    svec = jnp.zeros((1, 1, _G * _K), jnp.float32)
    ivec = jnp.zeros((1, 1, _G * _K), jnp.int32)
    bvec = jnp.zeros((1, 1, _G * _K), jnp.int32)
    for g in range(_G):
        b0 = bias_ref[0, 0, g]
        b1 = bias_ref[0, 0, g + _G]
        t0 = lprobs_ref[0, g] - pen_ref[...]
        t1 = lprobs_ref[0, g + _G] - pen_ref[...]
        rm0 = jnp.max(t0, axis=1, keepdims=True) + b0
        rm1 = jnp.max(t1, axis=1, keepdims=True) + b1
        picks = []
        toks = []
        for i in range(_K):
            m0 = jnp.max(rm0)
            m1 = jnp.max(rm1)
            m = jnp.maximum(m0, m1)
            any0 = m0 >= m1
            sub = jnp.where(any0, 0, 1).astype(jnp.int32)
            rmw = jnp.where(any0, rm0, rm1)
            r = jnp.min(jnp.where(rmw == m, riota, BIG))
            bsel = jnp.int32(g) + sub * _G
            raw = lprobs_ref[0, bsel, pl.ds(r, 1), :]
            prow = pen_ref[pl.ds(r, 1), :]
            row = (raw - prow) + jnp.where(any0, b0, b1)
            for ps, pr, pc in picks:
                same = jnp.logical_and(ps == sub, pr == r)
                row = jnp.where(jnp.logical_and(col == pc, same), NEGF, row)
            c_ = jnp.min(jnp.where(row == m, col, BIG))
            tok = r * C + c_
            slot = oiota == (i * _G + g)
            svec = jnp.where(slot, m, svec)
            ivec = jnp.where(slot, tok, ivec)
            bvec = jnp.where(slot, sub * _G + g, bvec)
            row = jnp.where(col == c_, NEGF, row)
            nm = jnp.max(row)
            rm0 = jnp.where(jnp.logical_and(any0, riota == r), nm, rm0)
            rm1 = jnp.where(
                jnp.logical_and(jnp.logical_not(any0), riota == r), nm, rm1
            )
            picks.append((sub, r, c_))
            toks.append(tok)
        if g < _G - 1:
            cnt = (f == toks[0]).astype(jnp.float32)
            for t in toks[1:]:
                cnt = cnt + (f == t).astype(jnp.float32)
            pen_ref[...] = pen_ref[...] + _ALPHA * cnt
    outs_ref[...] = svec
    outi_ref[...] = ivec
    outb_ref[...] = bvec


def kernel(step, lprobs, scores):
    bsz, beam, vocab = lprobs.shape
    C = vocab // _R
    lp4 = lprobs.reshape(bsz, beam, _R, C)
    bias = jax.lax.dynamic_slice_in_dim(scores, step - 1, 1, axis=2)
    bias = bias.reshape(bsz, 1, beam)
    out_shapes = [
        jax.ShapeDtypeStruct((bsz, 1, _G * _K), jnp.float32),
        jax.ShapeDtypeStruct((bsz, 1, _G * _K), jnp.int32),
        jax.ShapeDtypeStruct((bsz, 1, _G * _K), jnp.int32),
    ]
    in_specs = [
        pl.BlockSpec((1, beam, _R, C), lambda b: (b, 0, 0, 0)),
        pl.BlockSpec((1, 1, beam), lambda b: (b, 0, 0)),
    ]
    out_specs = [pl.BlockSpec((1, 1, _G * _K), lambda b: (b, 0, 0))] * 3
    sc, idx, bm = pl.pallas_call(
        _tc_body,
        grid=(bsz,),
        in_specs=in_specs,
        out_specs=out_specs,
        out_shape=out_shapes,
        scratch_shapes=[pltpu.VMEM((_R, C), jnp.float32)],
        compiler_params=pltpu.CompilerParams(
            dimension_semantics=("parallel",)
        ),
    )(lp4, bias)
    shp = (bsz, _G * _K)
    return sc.reshape(shp), idx.reshape(shp), bm.reshape(shp)


# speedup vs baseline: 1.1428x; 1.1428x over previous
import jax
import jax.numpy as jnp
from jax.experimental import pallas as pl
from jax.experimental.pallas import tpu as pltpu

_G = 4
_ALPHA = 0.5
_K = 4
_NEG = -3.0e38
_R = 8
_BB = 2


def _tc_body(lprobs_ref, bias_ref, outs_ref, outi_ref, outb_ref):
    C = lprobs_ref.shape[-1]
    f = (
        jax.lax.broadcasted_iota(jnp.int32, (_R, C), 0) * C
        + jax.lax.broadcasted_iota(jnp.int32, (_R, C), 1)
    )
    BIG = jnp.int32(2**30)
    oiota = jax.lax.broadcasted_iota(jnp.int32, (1, 1, _G * _K), 2)
    for bb in range(_BB):
        pen = jnp.zeros((_R, C), jnp.float32)
        svec = jnp.zeros((1, 1, _G * _K), jnp.float32)
        ivec = jnp.zeros((1, 1, _G * _K), jnp.int32)
        bvec = jnp.zeros((1, 1, _G * _K), jnp.int32)
        for g in range(_G):
            b0 = bias_ref[bb, 0, g]
            b1 = bias_ref[bb, 0, g + _G]
            lp0 = lprobs_ref[bb, g] + b0 - pen
            lp1 = lprobs_ref[bb, g + _G] + b1 - pen
            toks = []
            for i in range(_K):
                m0 = jnp.max(lp0)
                m1 = jnp.max(lp1)
                m = jnp.maximum(m0, m1)
                any0 = m0 >= m1
                i0 = jnp.min(jnp.where(lp0 == m, f, BIG))
                i1 = jnp.min(jnp.where(lp1 == m, f, BIG))
                tok = jnp.where(any0, i0, i1).astype(jnp.int32)
                sub = jnp.where(any0, 0, 1).astype(jnp.int32)
                slot = oiota == (i * _G + g)
                svec = jnp.where(slot, m, svec)
                ivec = jnp.where(slot, tok, ivec)
                bvec = jnp.where(slot, sub * _G + g, bvec)
                hit = f == tok
                lp0 = jnp.where(hit & any0, _NEG, lp0)
                lp1 = jnp.where(hit & jnp.logical_not(any0), _NEG, lp1)
                toks.append(tok)
            if g < _G - 1:
                for t in toks:
                    pen = pen + _ALPHA * (f == t).astype(jnp.float32)
        outs_ref[bb] = svec[0]
        outi_ref[bb] = ivec[0]
        outb_ref[bb] = bvec[0]


def kernel(step, lprobs, scores):
    bsz, beam, vocab = lprobs.shape
    C = vocab // _R
    lp4 = lprobs.reshape(bsz, beam, _R, C)
    bias = jax.lax.dynamic_slice_in_dim(scores, step - 1, 1, axis=2)
    bias = bias.reshape(bsz, 1, beam)
    out_shapes = [
        jax.ShapeDtypeStruct((bsz, 1, _G * _K), jnp.float32),
        jax.ShapeDtypeStruct((bsz, 1, _G * _K), jnp.int32),
        jax.ShapeDtypeStruct((bsz, 1, _G * _K), jnp.int32),
    ]
    in_specs = [
        pl.BlockSpec((_BB, beam, _R, C), lambda b: (b, 0, 0, 0)),
        pl.BlockSpec((_BB, 1, beam), lambda b: (b, 0, 0)),
    ]
    out_specs = [pl.BlockSpec((_BB, 1, _G * _K), lambda b: (b, 0, 0))] * 3
    sc, idx, bm = pl.pallas_call(
        _tc_body,
        grid=(bsz // _BB,),
        in_specs=in_specs,
        out_specs=out_specs,
        out_shape=out_shapes,
        compiler_params=pltpu.CompilerParams(
            dimension_semantics=("parallel",)
        ),
    )(lp4, bias)
    shp = (bsz, _G * _K)
    return sc.reshape(shp), idx.reshape(shp), bm.reshape(shp)
